# Initial kernel scaffold; baseline (speedup 1.0000x reference)
import jax
import jax.numpy as jnp
from jax.experimental import pallas as pl


def kernel(x, edge_index, edge_type, edge_attr, batch, params):
    raise NotImplementedError("write your pallas kernel here")


# speedup vs baseline: 1.2952x; 1.2952x over previous
import functools

import jax
import jax.numpy as jnp
from jax import lax
from jax.experimental import pallas as pl
from jax.experimental.pallas import tpu as pltpu
from jax.experimental.pallas import tpu_sc as plsc

HID = 128
EA = 4
NT = 5
NG = 64
N_NODES = 10000
N_EDGES = 320000

B_E = 256
E_PAD = 323584
NB = E_PAD // B_E
NW = 32
CHUNK = E_PAD // NW
G = 128
NTILES = CHUNK // G
N_MSG = 10112
TRASH = N_NODES
R_N = 400
NB_N = N_NODES // R_N


def _sc_gather(h, idx):
    mesh = plsc.VectorSubcoreMesh(core_axis_name="c", subcore_axis_name="s")

    @functools.partial(
        pl.kernel, mesh=mesh,
        out_type=jax.ShapeDtypeStruct((E_PAD, HID), jnp.float32),
        scratch_types=[
            pltpu.VMEM((G,), jnp.int32),
            pltpu.VMEM((G, HID), jnp.float32),
            pltpu.SemaphoreType.DMA,
        ],
    )
    def k(h_hbm, idx_hbm, out_hbm, idx_v, rows_v, sem):
        wid = lax.axis_index("s") * 2 + lax.axis_index("c")
        base = wid * CHUNK

        def body(t, carry):
            start = base + t * G
            pltpu.sync_copy(idx_hbm.at[pl.ds(start, G)], idx_v)
            pltpu.async_copy(h_hbm.at[idx_v], rows_v, sem).wait()
            pltpu.sync_copy(rows_v, out_hbm.at[pl.ds(start, G)])
            return carry

        lax.fori_loop(0, NTILES, body, 0)

    return k(h, idx)


def _sc_scatter(m, dst, zeros_msg):
    mesh = plsc.VectorSubcoreMesh(core_axis_name="c", subcore_axis_name="s")

    @functools.partial(
        pl.kernel, mesh=mesh,
        out_type=jax.ShapeDtypeStruct((2, N_MSG, HID), jnp.float32),
        scratch_types=[
            pltpu.VMEM((G,), jnp.int32),
            pltpu.VMEM((G, HID), jnp.float32),
            pltpu.VMEM_SHARED((N_MSG, HID), jnp.float32),
            pltpu.SemaphoreType.DMA,
        ],
    )
    def k(m_hbm, dst_hbm, z_hbm, out_hbm, idx_v, rows_v, acc, sem):
        cid = lax.axis_index("c")
        sid = lax.axis_index("s")
        wid = sid * 2 + cid
        base = wid * CHUNK

        @pl.when(sid == 0)
        def _():
            pltpu.sync_copy(z_hbm, acc)

        plsc.subcore_barrier()

        def body(t, carry):
            start = base + t * G
            pltpu.sync_copy(dst_hbm.at[pl.ds(start, G)], idx_v)
            pltpu.async_copy(m_hbm.at[pl.ds(start, G)], rows_v, sem).wait()
            pltpu.sync_copy(rows_v, acc.at[idx_v], add=True)
            return carry

        lax.fori_loop(0, NTILES, body, 0)
        plsc.subcore_barrier()
        stripe = N_MSG // 16
        pltpu.sync_copy(acc.at[pl.ds(sid * stripe, stripe)],
                        out_hbm.at[cid, pl.ds(sid * stripe, stripe)])

    return k(m, dst, zeros_msg)


def _mlp_blocks(hsrc, ea8, w1h, w1a, b1, w2, b2, btype):
    def kern(bt, hs, ea, w1h_r, w1a_r, b1_r, w2_r, b2_r, out):
        t = hs[...] @ w1h_r[0] + ea[...] @ w1a_r[0] + b1_r[0]
        t = jnp.maximum(t, 0.0)
        out[...] = t @ w2_r[0] + b2_r[0]

    return pl.pallas_call(
        kern,
        grid_spec=pltpu.PrefetchScalarGridSpec(
            num_scalar_prefetch=1,
            grid=(NB,),
            in_specs=[
                pl.BlockSpec((B_E, HID), lambda i, bt: (i, 0)),
                pl.BlockSpec((B_E, 8), lambda i, bt: (i, 0)),
                pl.BlockSpec((1, HID, HID), lambda i, bt: (bt[i], 0, 0)),
                pl.BlockSpec((1, 8, HID), lambda i, bt: (bt[i], 0, 0)),
                pl.BlockSpec((1, 1, HID), lambda i, bt: (bt[i], 0, 0)),
                pl.BlockSpec((1, HID, HID), lambda i, bt: (bt[i], 0, 0)),
                pl.BlockSpec((1, 1, HID), lambda i, bt: (bt[i], 0, 0)),
            ],
            out_specs=pl.BlockSpec((B_E, HID), lambda i, bt: (i, 0)),
        ),
        out_shape=jax.ShapeDtypeStruct((E_PAD, HID), jnp.float32),
        compiler_params=pltpu.CompilerParams(
            dimension_semantics=("arbitrary",)),
    )(btype, hsrc, ea8, w1h, w1a, b1, w2, b2)


def _gru_blocks(h, m0, m1, wihT, whhT, bih, bhh):
    def kern(h_r, m0_r, m1_r, wih_r, whh_r, bi_r, bh_r, out):
        msg = m0_r[...] + m1_r[...]
        gi = msg @ wih_r[...] + bi_r[...]
        gh = h_r[...] @ whh_r[...] + bh_r[...]
        r = jax.nn.sigmoid(gi[:, :HID] + gh[:, :HID])
        z = jax.nn.sigmoid(gi[:, HID:2 * HID] + gh[:, HID:2 * HID])
        n = jnp.tanh(gi[:, 2 * HID:] + r * gh[:, 2 * HID:])
        out[...] = (1.0 - z) * n + z * h_r[...]

    return pl.pallas_call(
        kern,
        grid=(NB_N,),
        in_specs=[
            pl.BlockSpec((R_N, HID), lambda i: (i, 0)),
            pl.BlockSpec((R_N, HID), lambda i: (i, 0)),
            pl.BlockSpec((R_N, HID), lambda i: (i, 0)),
            pl.BlockSpec((HID, 3 * HID), lambda i: (0, 0)),
            pl.BlockSpec((HID, 3 * HID), lambda i: (0, 0)),
            pl.BlockSpec((1, 3 * HID), lambda i: (0, 0)),
            pl.BlockSpec((1, 3 * HID), lambda i: (0, 0)),
        ],
        out_specs=pl.BlockSpec((R_N, HID), lambda i: (i, 0)),
        out_shape=jax.ShapeDtypeStruct((N_NODES, HID), jnp.float32),
        compiler_params=pltpu.CompilerParams(
            dimension_semantics=("arbitrary",)),
    )(h, m0, m1, wihT, whhT, bih, bhh)


def _readout(h, seg3d, w1T, b1, w2T, b2):
    def kern(h_r, seg_r, w1_r, b1_r, w2_r, b2_r, out, acc):
        i = pl.program_id(0)

        @pl.when(i == 0)
        def _():
            acc[...] = jnp.zeros_like(acc)

        seg = seg_r[0]
        row = lax.broadcasted_iota(jnp.int32, (NG, R_N), 0)
        onehot = (row == seg).astype(jnp.float32)
        acc[...] += onehot @ h_r[...]

        @pl.when(i == NB_N - 1)
        def _():
            g = jnp.maximum(acc[...] @ w1_r[...] + b1_r[...], 0.0)
            out[...] = g @ w2_r[...] + b2_r[0, 0]

    return pl.pallas_call(
        kern,
        grid=(NB_N,),
        in_specs=[
            pl.BlockSpec((R_N, HID), lambda i: (i, 0)),
            pl.BlockSpec((1, 1, R_N), lambda i: (i, 0, 0)),
            pl.BlockSpec((HID, HID), lambda i: (0, 0)),
            pl.BlockSpec((1, HID), lambda i: (0, 0)),
            pl.BlockSpec((HID, 1), lambda i: (0, 0)),
            pl.BlockSpec((1, 1), lambda i: (0, 0)),
        ],
        out_specs=pl.BlockSpec((NG, 1), lambda i: (0, 0)),
        out_shape=jax.ShapeDtypeStruct((NG, 1), jnp.float32),
        scratch_shapes=[pltpu.VMEM((NG, HID), jnp.float32)],
        compiler_params=pltpu.CompilerParams(
            dimension_semantics=("arbitrary",)),
    )(h, seg3d, w1T, b1, w2T, b2)


def _pack_mlps(mlps, ntypes):
    w1h = jnp.zeros((NT, HID, HID), jnp.float32)
    w1a = jnp.zeros((NT, 8, HID), jnp.float32)
    b1 = jnp.zeros((NT, 1, HID), jnp.float32)
    w2 = jnp.zeros((NT, HID, HID), jnp.float32)
    b2 = jnp.zeros((NT, 1, HID), jnp.float32)
    for t in range(ntypes):
        mp = mlps[t]
        w1h = w1h.at[t].set(mp["W1"][:, :HID].T)
        w1a = w1a.at[t, :EA].set(mp["W1"][:, HID:].T)
        b1 = b1.at[t, 0].set(mp["b1"])
        w2 = w2.at[t].set(mp["W2"].T)
        b2 = b2.at[t, 0].set(mp["b2"])
    return w1h, w1a, b1, w2, b2


def _pack_gru(g):
    return (g["Wih"].T, g["Whh"].T,
            g["bih"].reshape(1, 3 * HID), g["bhh"].reshape(1, 3 * HID))


def kernel(x, edge_index, edge_type, edge_attr, batch, params):
    src, dst = edge_index[0], edge_index[1]

    tids = jnp.arange(NT, dtype=jnp.int32)
    type_eq = edge_type[None, :] == tids[:, None]
    counts = type_eq.sum(axis=1).astype(jnp.int32)
    rank = (jnp.cumsum(type_eq, axis=1) - 1).astype(jnp.int32)
    rank = jnp.where(type_eq, rank, 0).sum(axis=0)
    padded = ((counts + B_E - 1) // B_E) * B_E
    cum = jnp.cumsum(padded)
    base = cum - padded
    pos = base[edge_type] + rank

    src_p = jnp.zeros((E_PAD,), jnp.int32).at[pos].set(src)
    dst_p = jnp.full((E_PAD,), TRASH, jnp.int32).at[pos].set(dst)
    ea8 = jnp.zeros((E_PAD, 8), jnp.float32)
    ea8 = ea8.at[pos].set(jnp.pad(edge_attr, ((0, 0), (0, 8 - EA))))
    btype = jnp.clip(
        jnp.searchsorted(cum, jnp.arange(NB, dtype=jnp.int32) * B_E,
                         side="right"),
        0, NT - 1).astype(jnp.int32)

    zeros_msg = jnp.zeros((N_MSG, HID), jnp.float32)
    seg3d = batch.astype(jnp.int32).reshape(NB_N, 1, R_N)

    def layer(h, mlp_pack, gru_pack):
        hsrc = _sc_gather(h, src_p)
        m = _mlp_blocks(hsrc, ea8, *mlp_pack, btype)
        msgs = _sc_scatter(m, dst_p, zeros_msg)
        m0 = lax.slice(msgs[0], (0, 0), (N_NODES, HID))
        m1 = lax.slice(msgs[1], (0, 0), (N_NODES, HID))
        return _gru_blocks(h, m0, m1, *gru_pack)

    bond_packs = [( _pack_mlps(lp["mlps"], 1), _pack_gru(lp["gru"]))
                  for lp in params["bond"]]
    spatial_packs = [(_pack_mlps(lp["mlps"], NT), _pack_gru(lp["gru"]))
                     for lp in params["spatial"]]

    def bond_branch(hh):
        for mp, gp in bond_packs:
            hh = layer(hh, mp, gp)
        return hh

    h = lax.cond(counts[0] > 0, bond_branch, lambda hh: hh, x)
    for mp, gp in spatial_packs:
        h = layer(h, mp, gp)

    r = params["readout"]
    out = _readout(h, seg3d, r["W1"].T, r["b1"].reshape(1, HID),
                   r["W2"].T, r["b2"].reshape(1, 1))
    return out.reshape(-1)


# speedup vs baseline: 2.3136x; 1.7863x over previous
import functools

import jax
import jax.numpy as jnp
from jax import lax
from jax.experimental import pallas as pl
from jax.experimental.pallas import tpu as pltpu
from jax.experimental.pallas import tpu_sc as plsc

HID = 128
EA = 4
NT = 5
NG = 64
N_NODES = 10000
N_EDGES = 320000

B_E = 256
NBLK = N_EDGES // B_E
NW = 32
CHUNK = N_EDGES // NW
SUB = 80
KSUB = 5
OUTER = SUB * KSUB
NTILES = CHUNK // OUTER
NTI = NW * NTILES
HROW = 5200
ACC_R = 5248
STRIPE = ACC_R // 16
R_N = 400
NB_N = N_NODES // R_N
PB = HROW // R_N

_MESH = dict(core_axis_name="c", subcore_axis_name="s")


def _sc_gather(h, idx2):
    @functools.partial(
        pl.kernel, mesh=plsc.VectorSubcoreMesh(**_MESH),
        out_type=jax.ShapeDtypeStruct((N_EDGES, HID), jnp.float32),
        scratch_types=[
            pltpu.VMEM((KSUB, SUB), jnp.int32),
            pltpu.VMEM((OUTER, HID), jnp.float32),
            pltpu.SemaphoreType.DMA,
        ],
    )
    def k(h_hbm, idx_hbm, out_hbm, idx_v, rows_v, sem):
        wid = lax.axis_index("s") * 2 + lax.axis_index("c")
        base = wid * CHUNK

        def body(t, carry):
            start = base + t * OUTER
            pltpu.sync_copy(idx_hbm.at[wid * NTILES + t], idx_v)
            cps = [pltpu.async_copy(h_hbm.at[idx_v.at[j]],
                                    rows_v.at[pl.ds(j * SUB, SUB)], sem)
                   for j in range(KSUB)]
            for cp in cps:
                cp.wait()
            pltpu.sync_copy(rows_v, out_hbm.at[pl.ds(start, OUTER)])
            return carry

        lax.fori_loop(0, NTILES, body, 0)

    return k(h, idx2)


def _sc_scatter(m, dst2, zeros_msg):
    @functools.partial(
        pl.kernel, mesh=plsc.VectorSubcoreMesh(**_MESH),
        out_type=jax.ShapeDtypeStruct((2, 2, ACC_R, HID), jnp.float32),
        scratch_types=[
            pltpu.VMEM((KSUB, SUB), jnp.int32),
            pltpu.VMEM((OUTER, HID), jnp.float32),
            pltpu.VMEM_SHARED((ACC_R, HID), jnp.float32),
            pltpu.SemaphoreType.DMA,
        ],
    )
    def k(m_hbm, dst_hbm, z_hbm, out_hbm, idx_v, rows_v, acc, sem):
        cid = lax.axis_index("c")
        sid = lax.axis_index("s")
        wid = sid * 2 + cid
        base = wid * CHUNK

        for p in range(2):
            pltpu.sync_copy(z_hbm.at[pl.ds(sid * STRIPE, STRIPE)],
                            acc.at[pl.ds(sid * STRIPE, STRIPE)])
            plsc.subcore_barrier()

            def body(t, carry):
                start = base + t * OUTER
                pltpu.sync_copy(dst_hbm.at[p, wid * NTILES + t], idx_v)
                pltpu.async_copy(m_hbm.at[pl.ds(start, OUTER)], rows_v,
                                 sem).wait()
                cps = [pltpu.async_copy(rows_v.at[pl.ds(j * SUB, SUB)],
                                        acc.at[idx_v.at[j]], sem, add=True)
                       for j in range(KSUB)]
                for cp in cps:
                    cp.wait()
                return carry

            lax.fori_loop(0, NTILES, body, 0)
            plsc.subcore_barrier()
            pltpu.sync_copy(acc.at[pl.ds(sid * STRIPE, STRIPE)],
                            out_hbm.at[p, cid, pl.ds(sid * STRIPE, STRIPE)])
            plsc.subcore_barrier()

    return k(m, dst2, zeros_msg)


def _mlp_blocks(hsrc, rec_s, w1h, w1a, b1, w2, b2, bt):
    def kern(bt_ref, hs_ref, rec_ref, w1h_r, w1a_r, b1_r, w2_r, b2_r, out):
        k = pl.program_id(0)
        t = bt_ref[k]
        ea = rec_ref[...][:, :EA]

        def mlp_t(tt):
            h1 = hs_ref[...] @ w1h_r[tt] + ea @ w1a_r[tt] + b1_r[tt]
            return jnp.maximum(h1, 0.0) @ w2_r[tt] + b2_r[tt]

        @pl.when(t >= 0)
        def _():
            out[...] = mlp_t(t)

        @pl.when(t == -1)
        def _():
            typef = rec_ref[...][:, 7:8]
            acc = jnp.zeros((B_E, HID), jnp.float32)
            for tt in range(NT):
                acc = acc + jnp.where(typef == float(tt), mlp_t(tt), 0.0)
            out[...] = acc

    return pl.pallas_call(
        kern,
        grid_spec=pltpu.PrefetchScalarGridSpec(
            num_scalar_prefetch=1,
            grid=(NBLK,),
            in_specs=[
                pl.BlockSpec((B_E, HID),
                             lambda i, bt: (jnp.where(bt[i] == -2, 0, i), 0)),
                pl.BlockSpec((B_E, 8),
                             lambda i, bt: (jnp.where(bt[i] == -2, 0, i), 0)),
                pl.BlockSpec((NT, HID, HID), lambda i, bt: (0, 0, 0)),
                pl.BlockSpec((NT, EA, HID), lambda i, bt: (0, 0, 0)),
                pl.BlockSpec((NT, 1, HID), lambda i, bt: (0, 0, 0)),
                pl.BlockSpec((NT, HID, HID), lambda i, bt: (0, 0, 0)),
                pl.BlockSpec((NT, 1, HID), lambda i, bt: (0, 0, 0)),
            ],
            out_specs=pl.BlockSpec(
                (B_E, HID), lambda i, bt: (jnp.where(bt[i] == -2, NBLK, i), 0)),
        ),
        out_shape=jax.ShapeDtypeStruct(((NBLK + 1) * B_E, HID), jnp.float32),
        compiler_params=pltpu.CompilerParams(
            dimension_semantics=("arbitrary",)),
    )(bt, hsrc, rec_s, w1h, w1a, b1, w2, b2)


def _gru_blocks(h, msgs, wihT, whhT, bih, bhh):
    def kern(h_r, m0_r, m1_r, wih_r, whh_r, bi_r, bh_r, out):
        msg = m0_r[0, 0] + m1_r[0, 0]
        gi = msg @ wih_r[...] + bi_r[...]
        gh = h_r[...] @ whh_r[...] + bh_r[...]
        r = jax.nn.sigmoid(gi[:, :HID] + gh[:, :HID])
        z = jax.nn.sigmoid(gi[:, HID:2 * HID] + gh[:, HID:2 * HID])
        n = jnp.tanh(gi[:, 2 * HID:] + r * gh[:, 2 * HID:])
        out[...] = (1.0 - z) * n + z * h_r[...]

    return pl.pallas_call(
        kern,
        grid=(NB_N,),
        in_specs=[
            pl.BlockSpec((R_N, HID), lambda i: (i, 0)),
            pl.BlockSpec((1, 1, R_N, HID), lambda i: (i // PB, 0, i % PB, 0)),
            pl.BlockSpec((1, 1, R_N, HID), lambda i: (i // PB, 1, i % PB, 0)),
            pl.BlockSpec((HID, 3 * HID), lambda i: (0, 0)),
            pl.BlockSpec((HID, 3 * HID), lambda i: (0, 0)),
            pl.BlockSpec((1, 3 * HID), lambda i: (0, 0)),
            pl.BlockSpec((1, 3 * HID), lambda i: (0, 0)),
        ],
        out_specs=pl.BlockSpec((R_N, HID), lambda i: (i, 0)),
        out_shape=jax.ShapeDtypeStruct((N_NODES, HID), jnp.float32),
        compiler_params=pltpu.CompilerParams(
            dimension_semantics=("arbitrary",)),
    )(h, msgs, msgs, wihT, whhT, bih, bhh)


def _readout(h, seg3d, w1T, b1, w2T, b2):
    def kern(h_r, seg_r, w1_r, b1_r, w2_r, b2_r, out, acc):
        i = pl.program_id(0)

        @pl.when(i == 0)
        def _():
            acc[...] = jnp.zeros_like(acc)

        seg = seg_r[0]
        row = lax.broadcasted_iota(jnp.int32, (NG, R_N), 0)
        onehot = (row == seg).astype(jnp.float32)
        acc[...] += onehot @ h_r[...]

        @pl.when(i == NB_N - 1)
        def _():
            g = jnp.maximum(acc[...] @ w1_r[...] + b1_r[...], 0.0)
            out[...] = g @ w2_r[...] + b2_r[0, 0]

    return pl.pallas_call(
        kern,
        grid=(NB_N,),
        in_specs=[
            pl.BlockSpec((R_N, HID), lambda i: (i, 0)),
            pl.BlockSpec((1, 1, R_N), lambda i: (i, 0, 0)),
            pl.BlockSpec((HID, HID), lambda i: (0, 0)),
            pl.BlockSpec((1, HID), lambda i: (0, 0)),
            pl.BlockSpec((HID, 1), lambda i: (0, 0)),
            pl.BlockSpec((1, 1), lambda i: (0, 0)),
        ],
        out_specs=pl.BlockSpec((NG, 1), lambda i: (0, 0)),
        out_shape=jax.ShapeDtypeStruct((NG, 1), jnp.float32),
        scratch_shapes=[pltpu.VMEM((NG, HID), jnp.float32)],
        compiler_params=pltpu.CompilerParams(
            dimension_semantics=("arbitrary",)),
    )(h, seg3d, w1T, b1, w2T, b2)


def _pack_mlps(mlps, ntypes):
    w1h = jnp.zeros((NT, HID, HID), jnp.float32)
    w1a = jnp.zeros((NT, EA, HID), jnp.float32)
    b1 = jnp.zeros((NT, 1, HID), jnp.float32)
    w2 = jnp.zeros((NT, HID, HID), jnp.float32)
    b2 = jnp.zeros((NT, 1, HID), jnp.float32)
    for t in range(ntypes):
        mp = mlps[t]
        w1h = w1h.at[t].set(mp["W1"][:, :HID].T)
        w1a = w1a.at[t].set(mp["W1"][:, HID:].T)
        b1 = b1.at[t, 0].set(mp["b1"])
        w2 = w2.at[t].set(mp["W2"].T)
        b2 = b2.at[t, 0].set(mp["b2"])
    return w1h, w1a, b1, w2, b2


def _pack_gru(g):
    return (g["Wih"].T, g["Whh"].T,
            g["bih"].reshape(1, 3 * HID), g["bhh"].reshape(1, 3 * HID))


def kernel(x, edge_index, edge_type, edge_attr, batch, params):
    src, dst = edge_index[0], edge_index[1]
    et = edge_type.astype(jnp.int32)

    tids = jnp.arange(NT, dtype=jnp.int32)
    type_eq = et[None, :] == tids[:, None]
    counts = type_eq.sum(axis=1).astype(jnp.int32)
    rank = (jnp.cumsum(type_eq, axis=1) - 1).astype(jnp.int32)
    rank = jnp.where(type_eq, rank, 0).sum(axis=0)
    cum = jnp.cumsum(counts)
    base = cum - counts
    pos = base[et] + rank

    as_f = lambda a: lax.bitcast_convert_type(a.astype(jnp.int32), jnp.float32)
    rec = jnp.concatenate(
        [edge_attr, as_f(src)[:, None], as_f(dst)[:, None],
         as_f(dst)[:, None], et.astype(jnp.float32)[:, None]], axis=1)
    inv = jnp.zeros((N_EDGES,), jnp.int32).at[pos].set(
        jnp.arange(N_EDGES, dtype=jnp.int32))
    rec_s = jnp.take(rec, inv, axis=0)

    as_i = lambda a: lax.bitcast_convert_type(a, jnp.int32)
    src2 = as_i(rec_s[:, 4]).reshape(NTI, KSUB, SUB)
    dst_s = as_i(rec_s[:, 5])
    type_s = rec_s[:, 7]

    trash_l = HROW + (jnp.arange(N_EDGES, dtype=jnp.int32) // CHUNK) % 16

    def _local_dst(valid):
        outs = []
        for p in range(2):
            lo = p * HROW
            inr = valid & (dst_s >= lo) & (dst_s < lo + HROW)
            outs.append(jnp.where(inr, dst_s - lo, trash_l))
        return jnp.stack(outs).reshape(2, NTI, KSUB, SUB)

    dst2 = _local_dst(jnp.ones((N_EDGES,), jnp.bool_))
    dstb2 = _local_dst(type_s == 0.0)

    blo = jnp.arange(NBLK, dtype=jnp.int32) * B_E
    t_lo = jnp.searchsorted(cum, blo, side="right").astype(jnp.int32)
    t_hi = jnp.searchsorted(cum, blo + (B_E - 1), side="right").astype(jnp.int32)
    bt_sp = jnp.where(t_lo == t_hi, t_lo, -1).astype(jnp.int32)
    c0 = counts[0]
    bt_bond = jnp.where(blo + B_E <= c0, 0,
                        jnp.where(blo < c0, -1, -2)).astype(jnp.int32)

    zeros_msg = jnp.zeros((ACC_R, HID), jnp.float32)
    seg3d = batch.astype(jnp.int32).reshape(NB_N, 1, R_N)

    def layer(h, mlp_pack, gru_pack, bt, d2):
        hsrc = _sc_gather(h, src2)
        m = _mlp_blocks(hsrc, rec_s, *mlp_pack, bt)
        msgs = _sc_scatter(m, d2, zeros_msg)
        return _gru_blocks(h, msgs, *gru_pack)

    bond_packs = [(_pack_mlps(lp["mlps"], 1), _pack_gru(lp["gru"]))
                  for lp in params["bond"]]
    spatial_packs = [(_pack_mlps(lp["mlps"], NT), _pack_gru(lp["gru"]))
                     for lp in params["spatial"]]

    def bond_branch(hh):
        for mp, gp in bond_packs:
            hh = layer(hh, mp, gp, bt_bond, dstb2)
        return hh

    h = lax.cond(c0 > 0, bond_branch, lambda hh: hh, x)
    for mp, gp in spatial_packs:
        h = layer(h, mp, gp, bt_sp, dst2)

    r = params["readout"]
    out = _readout(h, seg3d, r["W1"].T, r["b1"].reshape(1, HID),
                   r["W2"].T, r["b2"].reshape(1, 1))
    return out.reshape(-1)
